# baseline (speedup 1.0000x reference)
import functools

import jax
import jax.numpy as jnp
from jax import lax
from jax.experimental import pallas as pl
from jax.experimental.pallas import tpu as pltpu
from jax.experimental.pallas import tpu_sc as plsc

_NC = 2
_NS = 16
_L = 16
_CHUNK = 128
_NBUF = 8
_PG = 4


def _matmul(x, W):
    n, d_in = x.shape
    d_out = W.shape[1]
    dh = d_out // 2
    blk = 1000

    W2 = W.reshape(d_in, 2, dh).transpose(1, 0, 2)

    def body(x_ref, w_ref, o_ref):
        o_ref[0] = jnp.dot(x_ref[...], w_ref[0],
                           preferred_element_type=jnp.float32)

    return pl.pallas_call(
        body,
        grid=(n // blk, 2),
        in_specs=[pl.BlockSpec((blk, d_in), lambda i, j: (i, 0)),
                  pl.BlockSpec((1, d_in, dh), lambda i, j: (j, 0, 0))],
        out_specs=pl.BlockSpec((1, blk, dh), lambda i, j: (j, i, 0)),
        out_shape=jax.ShapeDtypeStruct((2, n, dh), jnp.float32),
    )(x, W2)


def _combine(partials, b):
    _, n, dh = partials.shape
    blk = 1000
    b2 = b.reshape(1, 2 * dh).astype(jnp.float32)

    def body(p_ref, b_ref, o_ref):
        o_ref[...] = (jnp.concatenate([p_ref[0], p_ref[1]], axis=-1)
                      + b_ref[...])

    return pl.pallas_call(
        body,
        grid=(n // blk,),
        in_specs=[pl.BlockSpec((2, blk, dh), lambda i: (0, i, 0)),
                  pl.BlockSpec((1, 2 * dh), lambda i: (0, 0))],
        out_specs=pl.BlockSpec((blk, 2 * dh), lambda i: (i, 0)),
        out_shape=jax.ShapeDtypeStruct((n, 2 * dh), jnp.float32),
    )(partials, b2)


def _sc_aggregate(support2, eidx, ew, k_real):
    _, n, dh = support2.shape
    rows_per_tile = -(-(-(-n // _NS)) // 8) * 8
    n_pad = _NS * rows_per_tile
    mesh = plsc.VectorSubcoreMesh(core_axis_name="c", subcore_axis_name="s")

    @functools.partial(
        pl.kernel,
        out_type=jax.ShapeDtypeStruct((_NC, n_pad, dh), jnp.float32),
        mesh=mesh,
        scratch_types=(
            [pltpu.VMEM((_CHUNK, dh), jnp.float32) for _ in range(_NBUF)]
            + [pltpu.VMEM((2, _CHUNK), jnp.int32) for _ in range(_NBUF)]
            + [pltpu.VMEM((_CHUNK,), jnp.float32) for _ in range(_NBUF)]
            + [pltpu.VMEM((_CHUNK,), jnp.int32) for _ in range(_NBUF)]
            + [pltpu.VMEM_SHARED((n_pad, dh), jnp.float32)]
            + [pltpu.SemaphoreType.DMA] * (3 * _NBUF)
        ),
        compiler_params=pltpu.CompilerParams(use_tc_tiling_on_sc=False),
    )
    def agg(s0_hbm, s1_hbm, eidx_hbm, ew_hbm, out_hbm, *refs):
        rows = refs[:_NBUF]
        slots = refs[_NBUF:2 * _NBUF]
        wslots = refs[2 * _NBUF:3 * _NBUF]
        dstbuf = refs[3 * _NBUF:4 * _NBUF]
        acc_sh = refs[4 * _NBUF]
        sems = refs[4 * _NBUF + 1:]
        gsems = sems[:_NBUF]
        ssems = sems[_NBUF:2 * _NBUF]
        isems = sems[2 * _NBUF:]

        c = lax.axis_index("c")
        s = lax.axis_index("s")

        def zero_row(r, carry):
            for dd in range(dh // _L):
                rows[0][r, pl.ds(dd * _L, _L)] = jnp.zeros((_L,), jnp.float32)
            return carry

        lax.fori_loop(0, _CHUNK, zero_row, 0)
        base = s * rows_per_tile
        nfull = rows_per_tile // _CHUNK
        for t in range(nfull):
            pltpu.sync_copy(rows[0],
                            acc_sh.at[pl.ds(base + t * _CHUNK, _CHUNK)])
        rem = rows_per_tile - nfull * _CHUNK
        if rem:
            pltpu.sync_copy(rows[0].at[pl.ds(0, rem)],
                            acc_sh.at[pl.ds(base + nfull * _CHUNK, rem)])
        plsc.subcore_barrier()

        def fire_idx(g, si):
            pltpu.async_copy(eidx_hbm.at[s, g], slots[si], isems[si])
            pltpu.async_copy(ew_hbm.at[s, g], wslots[si], isems[si])

        def wait_idx(si):
            pltpu.make_async_copy(eidx_hbm.at[0, 0], slots[si],
                                  isems[si]).wait()
            pltpu.make_async_copy(ew_hbm.at[0, 0], wslots[si],
                                  isems[si]).wait()

        def fire_gather(si, ri):
            src_row = slots[si].at[0]

            @pl.when(c == 0)
            def _():
                pltpu.async_copy(s0_hbm.at[src_row], rows[ri], gsems[ri])

            @pl.when(c == 1)
            def _():
                pltpu.async_copy(s1_hbm.at[src_row], rows[ri], gsems[ri])

        def wait_gather(ri):
            pltpu.make_async_copy(s0_hbm.at[pl.ds(0, _CHUNK)], rows[ri],
                                  gsems[ri]).wait()

        def scale(ri):
            buf = rows[ri]
            wrow = wslots[ri]

            def scale_group(grp, inner):
                wvec = wrow[pl.ds(grp * _L, _L)]
                for rr in range(_L):
                    ws = wvec[rr]
                    r = grp * _L + rr
                    for dd in range(dh // _L):
                        sl = pl.ds(dd * _L, _L)
                        buf[r, sl] = buf[r, sl] * ws
                return inner

            lax.fori_loop(0, _CHUNK // _L, scale_group, 0)

        def copy_dst(ri):
            for k in range(_CHUNK // _L):
                sl = pl.ds(k * _L, _L)
                dstbuf[ri][sl] = slots[ri][1, sl]

        def fire_scatter(ri):
            pltpu.async_copy(rows[ri], acc_sh.at[dstbuf[ri]], ssems[ri],
                             add=True)

        def wait_scatter(ri):
            pltpu.make_async_copy(s0_hbm.at[pl.ds(0, _CHUNK)], rows[ri],
                                  ssems[ri]).wait()

        for g in range(_PG + 2):
            fire_idx(g, g)
        for g in range(_PG):
            wait_idx(g)
            fire_gather(g, g)

        def step(g, ii):
            wait_gather(ii)
            copy_dst(ii)
            scale(ii)
            fire_scatter(ii)

            @pl.when(g >= _PG)
            def _():
                wait_scatter((ii + _PG) % _NBUF)

            fire_idx(g + _PG + 2, (ii + _PG + 2) % _NBUF)
            wait_idx((ii + _PG) % _NBUF)
            fire_gather((ii + _PG) % _NBUF, (ii + _PG) % _NBUF)

        def block_body(g0, carry):
            for ii in range(_NBUF):
                step(g0 * _NBUF + ii, ii)
            return carry

        lax.fori_loop(0, k_real // _NBUF, block_body, 0)

        for j in range(_PG):
            wait_gather(j % _NBUF)
            wait_scatter((k_real - _PG + j) % _NBUF)
        wait_idx((_PG) % _NBUF)
        wait_idx((_PG + 1) % _NBUF)
        plsc.subcore_barrier()

        pltpu.sync_copy(acc_sh.at[pl.ds(base, rows_per_tile)],
                        out_hbm.at[c, pl.ds(base, rows_per_tile)])

    return agg(support2[0], support2[1], eidx, ew)[:, :n, :]


def kernel(input, edge_index, edge_weight, W, b):
    n = input.shape[0]
    e = edge_weight.shape[0]
    per_tile = -(-e // _NS)
    k_real = -(-(-(-per_tile // _CHUNK)) // _NBUF) * _NBUF
    k_alloc = k_real + _PG + 2
    slots = k_alloc * _CHUNK

    def to_tiles(a):
        a = jnp.pad(a, (0, _NS * per_tile - e)).reshape(_NS, per_tile)
        a = jnp.pad(a, ((0, 0), (0, slots - per_tile)))
        return a.reshape(_NS, k_alloc, _CHUNK)

    eidx = jnp.stack([to_tiles(edge_index[0]), to_tiles(edge_index[1])],
                     axis=2)
    ew = to_tiles(edge_weight)

    support2 = _matmul(input.astype(jnp.float32), W.astype(jnp.float32))
    partials = _sc_aggregate(support2, eidx, ew, k_real)
    return _combine(partials, b)


# speedup vs baseline: 1.3220x; 1.3220x over previous
import functools

import jax
import jax.numpy as jnp
from jax import lax
from jax.experimental import pallas as pl
from jax.experimental.pallas import tpu as pltpu
from jax.experimental.pallas import tpu_sc as plsc

_NC = 2
_NS = 16
_L = 16
_CHUNK = 64
_NROW = 3


def _matmul(x, W):
    n, d_in = x.shape
    d_out = W.shape[1]
    blk = 1000

    def body(x_ref, w_ref, o_ref):
        o_ref[...] = jnp.dot(x_ref[...], w_ref[...],
                             preferred_element_type=jnp.float32)

    return pl.pallas_call(
        body,
        grid=(n // blk,),
        in_specs=[pl.BlockSpec((blk, d_in), lambda i: (i, 0)),
                  pl.BlockSpec((d_in, d_out), lambda i: (0, 0))],
        out_specs=pl.BlockSpec((blk, d_out), lambda i: (i, 0)),
        out_shape=jax.ShapeDtypeStruct((n, d_out), jnp.float32),
    )(x, W)


def _combine(partials, b):
    _, n, d = partials.shape
    blk = 1000
    b2 = b.reshape(1, d).astype(jnp.float32)

    def body(p_ref, b_ref, o_ref):
        o_ref[...] = p_ref[0] + p_ref[1] + b_ref[...]

    return pl.pallas_call(
        body,
        grid=(n // blk,),
        in_specs=[pl.BlockSpec((2, blk, d), lambda i: (0, i, 0)),
                  pl.BlockSpec((1, d), lambda i: (0, 0))],
        out_specs=pl.BlockSpec((blk, d), lambda i: (i, 0)),
        out_shape=jax.ShapeDtypeStruct((n, d), jnp.float32),
    )(partials, b2)


def _sc_aggregate(support, packed3, w3, k_real):
    n, d = support.shape
    k2 = packed3.shape[1]
    rows_per_tile = -(-(-(-n // _NS)) // 8) * 8
    n_pad = _NS * rows_per_tile
    mesh = plsc.VectorSubcoreMesh(core_axis_name="c", subcore_axis_name="s")

    @functools.partial(
        pl.kernel,
        out_type=jax.ShapeDtypeStruct((_NC, n_pad, d), jnp.float32),
        mesh=mesh,
        scratch_types=(
            [pltpu.VMEM((_CHUNK, d), jnp.float32) for _ in range(_NROW)]
            + [pltpu.VMEM((_CHUNK,), jnp.int32) for _ in range(_NROW)]
            + [pltpu.VMEM((_CHUNK,), jnp.int32) for _ in range(_NROW)]
            + [pltpu.VMEM((k2, 128), jnp.int32),
               pltpu.VMEM((k2, 128), jnp.float32),
               pltpu.VMEM_SHARED((n_pad, d), jnp.float32)]
            + [pltpu.SemaphoreType.DMA] * (2 * _NROW)
        ),
    )
    def agg(support_hbm, packed_hbm, w_hbm, out_hbm, *refs):
        rows = refs[:_NROW]
        sidx = refs[_NROW:2 * _NROW]
        dbuf = refs[2 * _NROW:3 * _NROW]
        packed_v = refs[3 * _NROW]
        w_v = refs[3 * _NROW + 1]
        acc_sh = refs[3 * _NROW + 2]
        sems = refs[3 * _NROW + 3:]
        gsems = sems[:_NROW]
        ssems = sems[_NROW:]

        c = lax.axis_index("c")
        s = lax.axis_index("s")
        wid = s * _NC + c

        def zero_row(r, carry):
            for dd in range(d // _L):
                rows[0][r, pl.ds(dd * _L, _L)] = jnp.zeros((_L,), jnp.float32)
            return carry

        lax.fori_loop(0, _CHUNK, zero_row, 0)
        base = s * rows_per_tile
        nfull = rows_per_tile // _CHUNK
        for t in range(nfull):
            pltpu.sync_copy(rows[0],
                            acc_sh.at[pl.ds(base + t * _CHUNK, _CHUNK)])
        rem = rows_per_tile - nfull * _CHUNK
        if rem:
            pltpu.sync_copy(rows[0].at[pl.ds(0, rem)],
                            acc_sh.at[pl.ds(base + nfull * _CHUNK, rem)])
        plsc.subcore_barrier()

        pltpu.sync_copy(packed_hbm.at[wid], packed_v)
        pltpu.sync_copy(w_hbm.at[wid], w_v)

        def decode(g, ri):
            gh = g >> 1
            off = (g & 1) * _CHUNK
            for grp in range(_CHUNK // _L):
                sl = pl.ds(grp * _L, _L)
                v = packed_v[gh, pl.ds(off + grp * _L, _L)]
                sidx[ri][sl] = v & 0xFFFF
                dbuf[ri][sl] = lax.shift_right_logical(v, 16)

        def fire_gather(ri):
            pltpu.async_copy(support_hbm.at[sidx[ri]], rows[ri], gsems[ri])

        def wait_gather(ri):
            pltpu.make_async_copy(support_hbm.at[pl.ds(0, _CHUNK)], rows[ri],
                                  gsems[ri]).wait()

        def scale(g, ri):
            buf = rows[ri]
            gh = g >> 1
            off = (g & 1) * _CHUNK

            def scale_group(grp, inner):
                wvec = w_v[gh, pl.ds(off + grp * _L, _L)]
                for rr in range(_L):
                    ws = wvec[rr]
                    r = grp * _L + rr
                    for dd in range(d // _L):
                        sl = pl.ds(dd * _L, _L)
                        buf[r, sl] = buf[r, sl] * ws
                return inner

            lax.fori_loop(0, _CHUNK // _L, scale_group, 0)

        def fire_scatter(ri):
            pltpu.async_copy(rows[ri], acc_sh.at[dbuf[ri]], ssems[ri],
                             add=True)

        def wait_scatter(ri):
            pltpu.make_async_copy(support_hbm.at[pl.ds(0, _CHUNK)], rows[ri],
                                  ssems[ri]).wait()

        for g in range(2):
            decode(g, g)
            fire_gather(g)

        def step(g, ii):
            wait_gather(ii)
            scale(g, ii)
            fire_scatter(ii)
            ri_next = (ii + 2) % _NROW

            @pl.when(g >= 1)
            def _():
                wait_scatter(ri_next)

            decode(g + 2, ri_next)
            fire_gather(ri_next)

        def block_body(g0, carry):
            for ii in range(_NROW):
                step(g0 * _NROW + ii, ii)
            return carry

        lax.fori_loop(0, k_real // _NROW, block_body, 0)

        wait_gather(k_real % _NROW)
        wait_gather((k_real + 1) % _NROW)
        wait_scatter((k_real - 1) % _NROW)
        plsc.subcore_barrier()

        pltpu.sync_copy(acc_sh.at[pl.ds(base, rows_per_tile)],
                        out_hbm.at[c, pl.ds(base, rows_per_tile)])

    return agg(support, packed3, w3)[:, :n, :]


def kernel(input, edge_index, edge_weight, W, b):
    n = input.shape[0]
    e = edge_weight.shape[0]
    nw = _NC * _NS
    per_tile = -(-e // nw)
    k_real = -(-(-(-per_tile // _CHUNK)) // _NROW) * _NROW
    k_alloc = k_real + 2 + (k_real % 2)
    slots = k_alloc * _CHUNK

    def to_tiles(a):
        a = jnp.pad(a, (0, nw * per_tile - e)).reshape(nw, per_tile)
        a = jnp.pad(a, ((0, 0), (0, slots - per_tile)))
        return a.reshape(nw, slots // 128, 128)

    packed3 = to_tiles((edge_index[1] << 16) | edge_index[0])
    w3 = to_tiles(edge_weight)

    support = _matmul(input.astype(jnp.float32), W.astype(jnp.float32))
    partials = _sc_aggregate(support, packed3, w3, k_real)
    return _combine(partials, b)


# speedup vs baseline: 1.7689x; 1.3380x over previous
import functools

import jax
import jax.numpy as jnp
from jax import lax
from jax.experimental import pallas as pl
from jax.experimental.pallas import tpu as pltpu
from jax.experimental.pallas import tpu_sc as plsc

_NC = 2
_NS = 16
_L = 16
_CHUNK = 128


def _matmul(x, W):
    n, d_in = x.shape
    d_out = W.shape[1]
    blk = 1000

    def body(x_ref, w_ref, o_ref):
        o_ref[...] = jnp.dot(x_ref[...], w_ref[...],
                             preferred_element_type=jnp.float32)

    return pl.pallas_call(
        body,
        grid=(n // blk,),
        in_specs=[pl.BlockSpec((blk, d_in), lambda i: (i, 0)),
                  pl.BlockSpec((d_in, d_out), lambda i: (0, 0))],
        out_specs=pl.BlockSpec((blk, d_out), lambda i: (i, 0)),
        out_shape=jax.ShapeDtypeStruct((n, d_out), jnp.float32),
    )(x, W)


def _combine(partials, b):
    _, n, d = partials.shape
    blk = 1000
    b2 = b.reshape(1, d).astype(jnp.float32)

    def body(p_ref, b_ref, o_ref):
        o_ref[...] = p_ref[0] + p_ref[1] + b_ref[...]

    return pl.pallas_call(
        body,
        grid=(n // blk,),
        in_specs=[pl.BlockSpec((2, blk, d), lambda i: (0, i, 0)),
                  pl.BlockSpec((1, d), lambda i: (0, 0))],
        out_specs=pl.BlockSpec((blk, d), lambda i: (i, 0)),
        out_shape=jax.ShapeDtypeStruct((n, d), jnp.float32),
    )(partials, b2)


def _sc_aggregate(support, src3, dst3, w3):
    n, d = support.shape
    k_chunks = src3.shape[1]
    rows_per_tile = -(-(-(-n // _NS)) // 8) * 8
    n_pad = _NS * rows_per_tile
    mesh = plsc.VectorSubcoreMesh(core_axis_name="c", subcore_axis_name="s")

    @functools.partial(
        pl.kernel,
        out_type=jax.ShapeDtypeStruct((_NC, n_pad, d), jnp.float32),
        mesh=mesh,
        scratch_types=[
            pltpu.VMEM((k_chunks, _CHUNK), jnp.int32),
            pltpu.VMEM((k_chunks, _CHUNK), jnp.int32),
            pltpu.VMEM((k_chunks, _CHUNK), jnp.float32),
            pltpu.VMEM((_CHUNK, d), jnp.float32),
            pltpu.VMEM_SHARED((n_pad, d), jnp.float32),
            pltpu.SemaphoreType.DMA,
        ],
    )
    def agg(support_hbm, src_hbm, dst_hbm, w_hbm, out_hbm,
            src_v, dst_v, w_v, rows_v, acc_sh, sem):
        c = lax.axis_index("c")
        s = lax.axis_index("s")
        wid = s * _NC + c

        def zero_row(r, carry):
            for dd in range(d // _L):
                rows_v[r, pl.ds(dd * _L, _L)] = jnp.zeros((_L,), jnp.float32)
            return carry

        lax.fori_loop(0, _CHUNK, zero_row, 0)
        base = s * rows_per_tile
        nfull = rows_per_tile // _CHUNK
        for t in range(nfull):
            pltpu.sync_copy(rows_v, acc_sh.at[pl.ds(base + t * _CHUNK, _CHUNK)])
        rem = rows_per_tile - nfull * _CHUNK
        if rem:
            pltpu.sync_copy(rows_v.at[pl.ds(0, rem)],
                            acc_sh.at[pl.ds(base + nfull * _CHUNK, rem)])
        plsc.subcore_barrier()

        pltpu.sync_copy(src_hbm.at[wid], src_v)
        pltpu.sync_copy(dst_hbm.at[wid], dst_v)
        pltpu.sync_copy(w_hbm.at[wid], w_v)

        def chunk_body(j, carry):
            pltpu.async_copy(support_hbm.at[src_v.at[j]], rows_v, sem).wait()

            def scale_group(g, inner):
                wvec = w_v[j, pl.ds(g * _L, _L)]
                for rr in range(_L):
                    ws = wvec[rr]
                    r = g * _L + rr
                    for dd in range(d // _L):
                        sl = pl.ds(dd * _L, _L)
                        rows_v[r, sl] = rows_v[r, sl] * ws
                return inner

            lax.fori_loop(0, _CHUNK // _L, scale_group, 0)
            pltpu.sync_copy(rows_v, acc_sh.at[dst_v.at[j]], add=True)
            return carry

        lax.fori_loop(0, k_chunks, chunk_body, 0)
        plsc.subcore_barrier()

        pltpu.sync_copy(acc_sh.at[pl.ds(base, rows_per_tile)],
                        out_hbm.at[c, pl.ds(base, rows_per_tile)])

    return agg(support, src3, dst3, w3)[:, :n, :]


def kernel(input, edge_index, edge_weight, W, b):
    n = input.shape[0]
    e = edge_weight.shape[0]
    nw = _NC * _NS
    per_tile = -(-e // nw)
    k_chunks = -(-per_tile // _CHUNK)
    slots = k_chunks * _CHUNK

    def to_tiles(a):
        a = jnp.pad(a, (0, nw * per_tile - e)).reshape(nw, per_tile)
        a = jnp.pad(a, ((0, 0), (0, slots - per_tile)))
        return a.reshape(nw, k_chunks, _CHUNK)

    src3 = to_tiles(edge_index[0])
    dst3 = to_tiles(edge_index[1])
    w3 = to_tiles(edge_weight)

    support = _matmul(input.astype(jnp.float32), W.astype(jnp.float32))
    partials = _sc_aggregate(support, src3, dst3, w3)
    return _combine(partials, b)
